# Initial kernel scaffold; baseline (speedup 1.0000x reference)
import jax
import jax.numpy as jnp
from jax.experimental import pallas as pl


def kernel(node_feature, hop_features, etype_f2v, etype_v2f, W_nm, b_nm, W_fm, b_fm, g_fm, bt_fm, W_v2v, b_v2v, g_v2v, bt_v2v, W_f2f, b_f2f, g_f2f, bt_f2f, W1_f2v, b1_f2v, Wm_f2v, bm_f2v, W2_f2v, b2_f2v, W1_v2f, b1_v2f, Wm_v2f, bm_v2f, W2_v2f, b2_v2f, Wc1, bc1, gc, btc, Wc2, bc2, nn_idx_f2v, nn_idx_v2f):
    raise NotImplementedError("write your pallas kernel here")


# speedup vs baseline: 5.9637x; 5.9637x over previous
import functools

import jax
import jax.numpy as jnp
from jax import lax
from jax.experimental import pallas as pl
from jax.experimental.pallas import tpu as pltpu
from jax.experimental.pallas import tpu_sc as plsc

N = 50000
F = 25000
K = 16
NE = 4
H = 64
EPS = 1e-5

BLK = 1000
GRID = N // BLK

SC_NC = 2
SC_NS = 16
SC_NW = SC_NC * SC_NS
E_EDGES = N * K
ROWS_PER_W = E_EDGES // SC_NW
SC_CHUNK = 1000
SC_ITERS = ROWS_PER_W // SC_CHUNK


def _k1_factor(hop_ref, wfm_ref, bfm_ref, gfm_ref, btfm_ref,
               w1_ref, b1_ref, wmh_ref, bm_ref, out_ref):
    y = jnp.dot(wfm_ref[...], hop_ref[...],
                preferred_element_type=jnp.float32) + bfm_ref[...]
    m = jnp.mean(y, axis=1, keepdims=True)
    v = jnp.mean((y - m) ** 2, axis=1, keepdims=True)
    nhop = jax.nn.relu((y - m) * lax.rsqrt(v + EPS) * gfm_ref[...]
                       + btfm_ref[...])
    h = jax.nn.relu(jnp.dot(w1_ref[...], nhop,
                            preferred_element_type=jnp.float32) + b1_ref[...])
    out_ref[...] = jnp.dot(wmh_ref[...], h,
                           preferred_element_type=jnp.float32) + bm_ref[...]


def _k2_node(xt_ref, wnm_ref, bnm_ref, wv2v_ref, bv2v_ref,
             nnode_ref, y2_ref, stats_ref):
    nn = jax.nn.relu(
        lax.dot_general(xt_ref[...], wnm_ref[...], (((1,), (1,)), ((), ())),
                        preferred_element_type=jnp.float32) + bnm_ref[...])
    y2 = lax.dot_general(nn, wv2v_ref[...], (((1,), (1,)), ((), ())),
                         preferred_element_type=jnp.float32) + bv2v_ref[...]
    nnode_ref[...] = nn
    y2_ref[...] = y2
    st = jnp.concatenate([jnp.sum(y2, axis=0, keepdims=True),
                          jnp.sum(y2 * y2, axis=0, keepdims=True)], axis=0)

    @pl.when(pl.program_id(0) == 0)
    def _():
        stats_ref[...] = st

    @pl.when(pl.program_id(0) != 0)
    def _():
        stats_ref[...] += st


def _sc_gather_body(table_ref, idx_ref, out_ref, idx_v, rows_v, sem):
    wid = lax.axis_index("s") * SC_NC + lax.axis_index("c")
    base = wid * ROWS_PER_W

    def body(j, carry):
        off = base + j * SC_CHUNK
        pltpu.sync_copy(idx_ref.at[pl.ds(off, SC_CHUNK)], idx_v)
        pltpu.async_copy(table_ref.at[idx_v], rows_v, sem).wait()
        pltpu.sync_copy(rows_v, out_ref.at[pl.ds(off, SC_CHUNK)])
        return carry

    lax.fori_loop(0, SC_ITERS, body, 0)


def _k4_combine(g3_ref, ef_ref, nnode_ref, y2_ref, stats_ref,
                wmet_ref, w2_ref, b2_ref, gv_ref, btv_ref,
                wc1_ref, bc1_ref, z_ref, zstats_ref):
    acc = None
    for k in range(K):
        e2 = jnp.dot(ef_ref[k], wmet_ref[...],
                     preferred_element_type=jnp.float32)
        mk = jax.nn.relu(g3_ref[k] + e2)
        acc = mk if acc is None else jnp.maximum(acc, mk)
    nv = lax.dot_general(acc, w2_ref[...], (((1,), (1,)), ((), ())),
                         preferred_element_type=jnp.float32) + b2_ref[...]
    st = stats_ref[...]
    m = st[0:1, :] / N
    v = st[1:2, :] / N - m * m
    inv = lax.rsqrt(v + EPS)
    y2 = y2_ref[...]
    nf = jax.nn.relu((y2 - m) * inv * gv_ref[...] + btv_ref[...]) + nv
    nnode2 = nnode_ref[...] + nf
    z = lax.dot_general(nnode2, wc1_ref[...], (((1,), (1,)), ((), ())),
                        preferred_element_type=jnp.float32) + bc1_ref[...]
    z_ref[...] = z
    st2 = jnp.concatenate([jnp.sum(z, axis=0, keepdims=True),
                           jnp.sum(z * z, axis=0, keepdims=True)], axis=0)

    @pl.when(pl.program_id(0) == 0)
    def _():
        zstats_ref[...] = st2

    @pl.when(pl.program_id(0) != 0)
    def _():
        zstats_ref[...] += st2


def _k5_final(z_ref, zstats_ref, gc_ref, btc_ref, wc2_ref, bc2_ref, out_ref):
    st = zstats_ref[...]
    m = st[0:1, :] / N
    v = st[1:2, :] / N - m * m
    inv = lax.rsqrt(v + EPS)
    hcls = jax.nn.relu((z_ref[...] - m) * inv * gc_ref[...] + btc_ref[...])
    o = jnp.sum(hcls * wc2_ref[...], axis=1, keepdims=True)
    out_ref[...] = o + bc2_ref[...]


def kernel(node_feature, hop_features, etype_f2v, etype_v2f, W_nm, b_nm, W_fm, b_fm, g_fm, bt_fm, W_v2v, b_v2v, g_v2v, bt_v2v, W_f2f, b_f2f, g_f2f, bt_f2f, W1_f2v, b1_f2v, Wm_f2v, bm_f2v, W2_f2v, b2_f2v, W1_v2f, b1_v2f, Wm_v2f, bm_v2f, W2_v2f, b2_v2f, Wc1, bc1, gc, btc, Wc2, bc2, nn_idx_f2v, nn_idx_v2f):
    xT = node_feature[0, :, :, 0].T
    hop = hop_features[0, 0, :, :, 0]
    idx = nn_idx_f2v[0, 0].astype(jnp.int32)
    idx_flat = idx.T.reshape(E_EDGES)
    efT = jnp.transpose(etype_f2v[0, 0], (2, 1, 0))

    row2 = lambda a: a.reshape(1, -1)

    g_tab = pl.pallas_call(
        _k1_factor,
        out_shape=jax.ShapeDtypeStruct((H, F), jnp.float32),
    )(hop, W_fm, b_fm.reshape(H, 1), g_fm.reshape(H, 1), bt_fm.reshape(H, 1),
      W1_f2v, b1_f2v.reshape(H, 1), Wm_f2v[:, :H], bm_f2v.reshape(H, 1))
    g_tabT = g_tab.T

    nnodeT, y2T, stats = pl.pallas_call(
        _k2_node,
        grid=(GRID,),
        in_specs=[
            pl.BlockSpec((BLK, 128), lambda i: (i, 0)),
            pl.BlockSpec((H, 128), lambda i: (0, 0)),
            pl.BlockSpec((1, H), lambda i: (0, 0)),
            pl.BlockSpec((H, H), lambda i: (0, 0)),
            pl.BlockSpec((1, H), lambda i: (0, 0)),
        ],
        out_specs=[
            pl.BlockSpec((BLK, H), lambda i: (i, 0)),
            pl.BlockSpec((BLK, H), lambda i: (i, 0)),
            pl.BlockSpec((2, H), lambda i: (0, 0)),
        ],
        out_shape=[
            jax.ShapeDtypeStruct((N, H), jnp.float32),
            jax.ShapeDtypeStruct((N, H), jnp.float32),
            jax.ShapeDtypeStruct((2, H), jnp.float32),
        ],
    )(xT, W_nm, row2(b_nm), W_v2v, row2(b_v2v))

    sc_gather = functools.partial(
        pl.kernel,
        mesh=plsc.VectorSubcoreMesh(core_axis_name="c", subcore_axis_name="s",
                                    num_cores=SC_NC, num_subcores=SC_NS),
        out_type=jax.ShapeDtypeStruct((E_EDGES, H), jnp.float32),
        scratch_types=[
            pltpu.VMEM((SC_CHUNK,), jnp.int32),
            pltpu.VMEM((SC_CHUNK, H), jnp.float32),
            pltpu.SemaphoreType.DMA,
        ],
        compiler_params=pltpu.CompilerParams(use_tc_tiling_on_sc=False),
    )(_sc_gather_body)
    g_rows = sc_gather(g_tabT, idx_flat)
    g3 = g_rows.reshape(K, N, H)

    zT, zstats = pl.pallas_call(
        _k4_combine,
        grid=(GRID,),
        in_specs=[
            pl.BlockSpec((K, BLK, H), lambda i: (0, i, 0)),
            pl.BlockSpec((K, BLK, NE), lambda i: (0, i, 0)),
            pl.BlockSpec((BLK, H), lambda i: (i, 0)),
            pl.BlockSpec((BLK, H), lambda i: (i, 0)),
            pl.BlockSpec((2, H), lambda i: (0, 0)),
            pl.BlockSpec((NE, H), lambda i: (0, 0)),
            pl.BlockSpec((H, H), lambda i: (0, 0)),
            pl.BlockSpec((1, H), lambda i: (0, 0)),
            pl.BlockSpec((1, H), lambda i: (0, 0)),
            pl.BlockSpec((1, H), lambda i: (0, 0)),
            pl.BlockSpec((128, H), lambda i: (0, 0)),
            pl.BlockSpec((1, 128), lambda i: (0, 0)),
        ],
        out_specs=[
            pl.BlockSpec((BLK, 128), lambda i: (i, 0)),
            pl.BlockSpec((2, 128), lambda i: (0, 0)),
        ],
        out_shape=[
            jax.ShapeDtypeStruct((N, 128), jnp.float32),
            jax.ShapeDtypeStruct((2, 128), jnp.float32),
        ],
    )(g3, efT, nnodeT, y2T, stats, Wm_f2v[:, H:].T, W2_f2v, row2(b2_f2v),
      row2(g_v2v), row2(bt_v2v), Wc1, row2(bc1))

    out = pl.pallas_call(
        _k5_final,
        grid=(GRID,),
        in_specs=[
            pl.BlockSpec((BLK, 128), lambda i: (i, 0)),
            pl.BlockSpec((2, 128), lambda i: (0, 0)),
            pl.BlockSpec((1, 128), lambda i: (0, 0)),
            pl.BlockSpec((1, 128), lambda i: (0, 0)),
            pl.BlockSpec((1, 128), lambda i: (0, 0)),
            pl.BlockSpec((1, 1), lambda i: (0, 0)),
        ],
        out_specs=pl.BlockSpec((BLK, 1), lambda i: (i, 0)),
        out_shape=jax.ShapeDtypeStruct((N, 1), jnp.float32),
    )(zT, zstats, row2(gc), row2(btc), Wc2, bc2.reshape(1, 1))

    return out.reshape(1, 1, N, 1)


# speedup vs baseline: 5.9787x; 1.0025x over previous
import functools

import jax
import jax.numpy as jnp
from jax import lax
from jax.experimental import pallas as pl
from jax.experimental.pallas import tpu as pltpu
from jax.experimental.pallas import tpu_sc as plsc

N = 50000
F = 25000
K = 16
NE = 4
H = 64
EPS = 1e-5

BLK = 1000
GRID = N // BLK

SC_NC = 2
SC_NS = 16
SC_NW = SC_NC * SC_NS
E_EDGES = N * K
ROWS_PER_W = E_EDGES // SC_NW
SC_CHUNK = 1000
SC_ITERS = ROWS_PER_W // SC_CHUNK


def _k1_factor(hop_ref, wfm_ref, bfm_ref, gfm_ref, btfm_ref,
               w1_ref, b1_ref, wmh_ref, bm_ref, out_ref):
    y = jnp.dot(wfm_ref[...], hop_ref[...],
                preferred_element_type=jnp.float32) + bfm_ref[...]
    m = jnp.mean(y, axis=1, keepdims=True)
    v = jnp.mean((y - m) ** 2, axis=1, keepdims=True)
    nhop = jax.nn.relu((y - m) * lax.rsqrt(v + EPS) * gfm_ref[...]
                       + btfm_ref[...])
    h = jax.nn.relu(jnp.dot(w1_ref[...], nhop,
                            preferred_element_type=jnp.float32) + b1_ref[...])
    out_ref[...] = jnp.dot(wmh_ref[...], h,
                           preferred_element_type=jnp.float32) + bm_ref[...]


def _k2_node(xt_ref, wnm_ref, bnm_ref, wv2v_ref, bv2v_ref,
             nnode_ref, y2_ref, stats_ref):
    nn = jax.nn.relu(
        lax.dot_general(xt_ref[...], wnm_ref[...], (((1,), (1,)), ((), ())),
                        preferred_element_type=jnp.float32) + bnm_ref[...])
    y2 = lax.dot_general(nn, wv2v_ref[...], (((1,), (1,)), ((), ())),
                         preferred_element_type=jnp.float32) + bv2v_ref[...]
    nnode_ref[...] = nn
    y2_ref[...] = y2
    st = jnp.concatenate([jnp.sum(y2, axis=0, keepdims=True),
                          jnp.sum(y2 * y2, axis=0, keepdims=True)], axis=0)

    @pl.when(pl.program_id(0) == 0)
    def _():
        stats_ref[...] = st

    @pl.when(pl.program_id(0) != 0)
    def _():
        stats_ref[...] += st


def _sc_gather_body(table_ref, idx_ref, out_ref, idx_v, rows_v, sem):
    wid = lax.axis_index("s") * SC_NC + lax.axis_index("c")
    base = wid * ROWS_PER_W

    def body(j, carry):
        off = base + j * SC_CHUNK
        pltpu.sync_copy(idx_ref.at[pl.ds(off, SC_CHUNK)], idx_v)
        pltpu.async_copy(table_ref.at[idx_v], rows_v, sem).wait()
        pltpu.sync_copy(rows_v, out_ref.at[pl.ds(off, SC_CHUNK)])
        return carry

    lax.fori_loop(0, SC_ITERS, body, 0)


def _k4_combine(g3_ref, ef_ref, nnode_ref, y2_ref, stats_ref,
                wmet_ref, w2_ref, b2_ref, gv_ref, btv_ref,
                wc1_ref, bc1_ref, z_ref, zstats_ref):
    acc = None
    for k in range(K):
        e2 = jnp.dot(ef_ref[k], wmet_ref[...],
                     preferred_element_type=jnp.float32)
        mk = jax.nn.relu(g3_ref[k] + e2)
        acc = mk if acc is None else jnp.maximum(acc, mk)
    nv = lax.dot_general(acc, w2_ref[...], (((1,), (1,)), ((), ())),
                         preferred_element_type=jnp.float32) + b2_ref[...]
    st = stats_ref[...]
    m = st[0:1, :] / N
    v = st[1:2, :] / N - m * m
    inv = lax.rsqrt(v + EPS)
    y2 = y2_ref[...]
    nf = jax.nn.relu((y2 - m) * inv * gv_ref[...] + btv_ref[...]) + nv
    nnode2 = nnode_ref[...] + nf
    z = lax.dot_general(nnode2, wc1_ref[...], (((1,), (1,)), ((), ())),
                        preferred_element_type=jnp.float32) + bc1_ref[...]
    z_ref[...] = z
    st2 = jnp.concatenate([jnp.sum(z, axis=0, keepdims=True),
                           jnp.sum(z * z, axis=0, keepdims=True)], axis=0)

    @pl.when(pl.program_id(0) == 0)
    def _():
        zstats_ref[...] = st2

    @pl.when(pl.program_id(0) != 0)
    def _():
        zstats_ref[...] += st2


def _k5_final(z_ref, zstats_ref, gc_ref, btc_ref, wc2_ref, bc2_ref, out_ref):
    st = zstats_ref[...]
    m = st[0:1, :] / N
    v = st[1:2, :] / N - m * m
    inv = lax.rsqrt(v + EPS)
    hcls = jax.nn.relu((z_ref[...] - m) * inv * gc_ref[...] + btc_ref[...])
    o = jnp.sum(hcls * wc2_ref[...], axis=1, keepdims=True)
    out_ref[...] = o + bc2_ref[...]


def kernel(node_feature, hop_features, etype_f2v, etype_v2f, W_nm, b_nm, W_fm, b_fm, g_fm, bt_fm, W_v2v, b_v2v, g_v2v, bt_v2v, W_f2f, b_f2f, g_f2f, bt_f2f, W1_f2v, b1_f2v, Wm_f2v, bm_f2v, W2_f2v, b2_f2v, W1_v2f, b1_v2f, Wm_v2f, bm_v2f, W2_v2f, b2_v2f, Wc1, bc1, gc, btc, Wc2, bc2, nn_idx_f2v, nn_idx_v2f):
    xT = node_feature[0, :, :, 0].T
    hop = hop_features[0, 0, :, :, 0]
    idx = nn_idx_f2v[0, 0].astype(jnp.int32)
    idx_flat = idx.T.reshape(E_EDGES)
    efT = jnp.transpose(etype_f2v[0, 0], (2, 1, 0))

    row2 = lambda a: a.reshape(1, -1)

    g_tab = pl.pallas_call(
        _k1_factor,
        out_shape=jax.ShapeDtypeStruct((H, F), jnp.float32),
    )(hop, W_fm, b_fm.reshape(H, 1), g_fm.reshape(H, 1), bt_fm.reshape(H, 1),
      W1_f2v, b1_f2v.reshape(H, 1), Wm_f2v[:, :H], bm_f2v.reshape(H, 1))
    g_tabT = jnp.pad(g_tab, ((0, H), (0, 0))).T

    nnodeT, y2T, stats = pl.pallas_call(
        _k2_node,
        grid=(GRID,),
        in_specs=[
            pl.BlockSpec((BLK, 128), lambda i: (i, 0)),
            pl.BlockSpec((H, 128), lambda i: (0, 0)),
            pl.BlockSpec((1, H), lambda i: (0, 0)),
            pl.BlockSpec((H, H), lambda i: (0, 0)),
            pl.BlockSpec((1, H), lambda i: (0, 0)),
        ],
        out_specs=[
            pl.BlockSpec((BLK, H), lambda i: (i, 0)),
            pl.BlockSpec((BLK, H), lambda i: (i, 0)),
            pl.BlockSpec((2, H), lambda i: (0, 0)),
        ],
        out_shape=[
            jax.ShapeDtypeStruct((N, H), jnp.float32),
            jax.ShapeDtypeStruct((N, H), jnp.float32),
            jax.ShapeDtypeStruct((2, H), jnp.float32),
        ],
    )(xT, W_nm, row2(b_nm), W_v2v, row2(b_v2v))

    sc_gather = functools.partial(
        pl.kernel,
        mesh=plsc.VectorSubcoreMesh(core_axis_name="c", subcore_axis_name="s",
                                    num_cores=SC_NC, num_subcores=SC_NS),
        out_type=jax.ShapeDtypeStruct((E_EDGES, 2 * H), jnp.float32),
        scratch_types=[
            pltpu.VMEM((SC_CHUNK,), jnp.int32),
            pltpu.VMEM((SC_CHUNK, 2 * H), jnp.float32),
            pltpu.SemaphoreType.DMA,
        ],
    )(_sc_gather_body)
    g_rows = sc_gather(g_tabT, idx_flat)
    g3 = g_rows.reshape(K, N, 2 * H)

    zT, zstats = pl.pallas_call(
        _k4_combine,
        grid=(GRID,),
        in_specs=[
            pl.BlockSpec((K, BLK, 2 * H), lambda i: (0, i, 0)),
            pl.BlockSpec((K, BLK, NE), lambda i: (0, i, 0)),
            pl.BlockSpec((BLK, H), lambda i: (i, 0)),
            pl.BlockSpec((BLK, H), lambda i: (i, 0)),
            pl.BlockSpec((2, H), lambda i: (0, 0)),
            pl.BlockSpec((NE, 2 * H), lambda i: (0, 0)),
            pl.BlockSpec((H, 2 * H), lambda i: (0, 0)),
            pl.BlockSpec((1, H), lambda i: (0, 0)),
            pl.BlockSpec((1, H), lambda i: (0, 0)),
            pl.BlockSpec((1, H), lambda i: (0, 0)),
            pl.BlockSpec((128, H), lambda i: (0, 0)),
            pl.BlockSpec((1, 128), lambda i: (0, 0)),
        ],
        out_specs=[
            pl.BlockSpec((BLK, 128), lambda i: (i, 0)),
            pl.BlockSpec((2, 128), lambda i: (0, 0)),
        ],
        out_shape=[
            jax.ShapeDtypeStruct((N, 128), jnp.float32),
            jax.ShapeDtypeStruct((2, 128), jnp.float32),
        ],
    )(g3, efT, nnodeT, y2T, stats,
      jnp.pad(Wm_f2v[:, H:].T, ((0, 0), (0, H))),
      jnp.pad(W2_f2v, ((0, 0), (0, H))), row2(b2_f2v),
      row2(g_v2v), row2(bt_v2v), Wc1, row2(bc1))

    out = pl.pallas_call(
        _k5_final,
        grid=(GRID,),
        in_specs=[
            pl.BlockSpec((BLK, 128), lambda i: (i, 0)),
            pl.BlockSpec((2, 128), lambda i: (0, 0)),
            pl.BlockSpec((1, 128), lambda i: (0, 0)),
            pl.BlockSpec((1, 128), lambda i: (0, 0)),
            pl.BlockSpec((1, 128), lambda i: (0, 0)),
            pl.BlockSpec((1, 1), lambda i: (0, 0)),
        ],
        out_specs=pl.BlockSpec((BLK, 1), lambda i: (i, 0)),
        out_shape=jax.ShapeDtypeStruct((N, 1), jnp.float32),
    )(zT, zstats, row2(gc), row2(btc), Wc2, bc2.reshape(1, 1))

    return out.reshape(1, 1, N, 1)


# speedup vs baseline: 11.9900x; 2.0055x over previous
import functools

import jax
import jax.numpy as jnp
from jax import lax
from jax.experimental import pallas as pl
from jax.experimental.pallas import tpu as pltpu
from jax.experimental.pallas import tpu_sc as plsc

N = 50000
F = 25000
K = 16
NE = 4
H = 64
EPS = 1e-5

BLK = 1000
GRID = N // BLK

SC_NC = 2
SC_NS = 16
SC_NW = SC_NC * SC_NS
E_EDGES = N * K
ROWS_PER_W = E_EDGES // SC_NW
SC_CHUNK = 200
N_SPLIT = 25200
N_HALVES = ((0, N_SPLIT), (N_SPLIT, N - N_SPLIT))


def _k1_factor(hop_ref, wfm_ref, bfm_ref, gfm_ref, btfm_ref,
               w1_ref, b1_ref, wmh_ref, bm_ref, out_ref):
    y = jnp.dot(wfm_ref[...], hop_ref[...],
                preferred_element_type=jnp.float32) + bfm_ref[...]
    m = jnp.mean(y, axis=1, keepdims=True)
    v = jnp.mean((y - m) ** 2, axis=1, keepdims=True)
    nhop = jax.nn.relu((y - m) * lax.rsqrt(v + EPS) * gfm_ref[...]
                       + btfm_ref[...])
    h = jax.nn.relu(jnp.dot(w1_ref[...], nhop,
                            preferred_element_type=jnp.float32) + b1_ref[...])
    out_ref[...] = lax.dot_general(h, wmh_ref[...], (((0,), (1,)), ((), ())),
                                   preferred_element_type=jnp.float32) + bm_ref[...]


def _k2_node(xt_ref, wnm_ref, bnm_ref, wv2v_ref, bv2v_ref,
             nnode_ref, y2_ref, stats_ref):
    nn = jax.nn.relu(
        lax.dot_general(xt_ref[...], wnm_ref[...], (((1,), (1,)), ((), ())),
                        preferred_element_type=jnp.float32) + bnm_ref[...])
    y2 = lax.dot_general(nn, wv2v_ref[...], (((1,), (1,)), ((), ())),
                         preferred_element_type=jnp.float32) + bv2v_ref[...]
    nnode_ref[...] = nn
    y2_ref[...] = y2
    st = jnp.concatenate([jnp.sum(y2, axis=0, keepdims=True),
                          jnp.sum(y2 * y2, axis=0, keepdims=True)], axis=0)

    @pl.when(pl.program_id(0) == 0)
    def _():
        stats_ref[...] = st

    @pl.when(pl.program_id(0) != 0)
    def _():
        stats_ref[...] += st


def _make_sc_gather_body(rows_per_w, iters):
    def _sc_gather_body(table_ref, idx_ref, out_ref,
                        idx0, idx1, rows0, rows1, sem0, sem1):
        wid = lax.axis_index("s") * SC_NC + lax.axis_index("c")
        base = wid * rows_per_w

        def _step(j, cur_idx, cur_rows, cur_sem, nxt_idx, nxt_rows, nxt_sem):
            @pl.when(j + 1 < iters)
            def _():
                off_n = base + (j + 1) * SC_CHUNK
                pltpu.sync_copy(idx_ref.at[pl.ds(off_n, SC_CHUNK)], nxt_idx)
                pltpu.async_copy(table_ref.at[nxt_idx], nxt_rows, nxt_sem)
            pltpu.make_async_copy(table_ref.at[cur_idx], cur_rows,
                                  cur_sem).wait()
            pltpu.sync_copy(cur_rows, out_ref.at[pl.ds(base + j * SC_CHUNK,
                                                       SC_CHUNK)])

        def body(j, carry):
            @pl.when(j % 2 == 0)
            def _():
                _step(j, idx0, rows0, sem0, idx1, rows1, sem1)

            @pl.when(j % 2 == 1)
            def _():
                _step(j, idx1, rows1, sem1, idx0, rows0, sem0)

            return carry

        pltpu.sync_copy(idx_ref.at[pl.ds(base, SC_CHUNK)], idx0)
        pltpu.async_copy(table_ref.at[idx0], rows0, sem0)
        lax.fori_loop(0, iters, body, 0)

    return _sc_gather_body


def _k4_combine(g3_ref, ef_ref, nnode_ref, y2_ref, stats_ref,
                wme_ref, w2_ref, b2_ref, gv_ref, btv_ref,
                wc1_ref, bc1_ref, z_ref, zstats_ref):
    e2 = lax.dot_general(ef_ref[:, 0], wme_ref[...], (((0,), (1,)), ((), ())),
                         preferred_element_type=jnp.float32)
    msg = jax.nn.relu(g3_ref[...] + e2)
    acc = jnp.max(msg, axis=1)
    nv = lax.dot_general(acc, w2_ref[...], (((1,), (1,)), ((), ())),
                         preferred_element_type=jnp.float32) + b2_ref[...]
    st = stats_ref[...]
    m = st[0:1, :] / N
    v = st[1:2, :] / N - m * m
    inv = lax.rsqrt(v + EPS)
    y2 = y2_ref[...]
    nf = jax.nn.relu((y2 - m) * inv * gv_ref[...] + btv_ref[...]) + nv
    nnode2 = nnode_ref[...] + nf
    z = lax.dot_general(nnode2, wc1_ref[...], (((1,), (1,)), ((), ())),
                        preferred_element_type=jnp.float32) + bc1_ref[...]
    z_ref[...] = z
    st2 = jnp.concatenate([jnp.sum(z, axis=0, keepdims=True),
                           jnp.sum(z * z, axis=0, keepdims=True)], axis=0)

    @pl.when(pl.program_id(0) == 0)
    def _():
        zstats_ref[...] = st2

    @pl.when(pl.program_id(0) != 0)
    def _():
        zstats_ref[...] += st2


def _k5_final(z_ref, zsta_ref, zstb_ref, gc_ref, btc_ref, wc2_ref, bc2_ref,
              out_ref):
    st = zsta_ref[...] + zstb_ref[...]
    m = st[0:1, :] / N
    v = st[1:2, :] / N - m * m
    inv = lax.rsqrt(v + EPS)
    hcls = jax.nn.relu((z_ref[...] - m) * inv * gc_ref[...] + btc_ref[...])
    o = jnp.sum(hcls * wc2_ref[...], axis=1, keepdims=True)
    out_ref[...] = o + bc2_ref[...]


def kernel(node_feature, hop_features, etype_f2v, etype_v2f, W_nm, b_nm, W_fm, b_fm, g_fm, bt_fm, W_v2v, b_v2v, g_v2v, bt_v2v, W_f2f, b_f2f, g_f2f, bt_f2f, W1_f2v, b1_f2v, Wm_f2v, bm_f2v, W2_f2v, b2_f2v, W1_v2f, b1_v2f, Wm_v2f, bm_v2f, W2_v2f, b2_v2f, Wc1, bc1, gc, btc, Wc2, bc2, nn_idx_f2v, nn_idx_v2f):
    xT = node_feature[0, :, :, 0].T
    hop = hop_features[0, 0, :, :, 0]
    idx = nn_idx_f2v[0, 0].astype(jnp.int32)
    idx_flat = idx.reshape(E_EDGES)
    ef = etype_f2v[0, 0]

    row2 = lambda a: a.reshape(1, -1)

    g_tabT = pl.pallas_call(
        _k1_factor,
        out_shape=jax.ShapeDtypeStruct((F, 2 * H), jnp.float32),
    )(hop, W_fm, b_fm.reshape(H, 1), g_fm.reshape(H, 1), bt_fm.reshape(H, 1),
      W1_f2v, b1_f2v.reshape(H, 1),
      jnp.pad(Wm_f2v[:, :H], ((0, H), (0, 0))),
      row2(jnp.pad(bm_f2v, (0, H))))

    nnodeT, y2T, stats = pl.pallas_call(
        _k2_node,
        grid=(GRID,),
        in_specs=[
            pl.BlockSpec((BLK, 128), lambda i: (i, 0)),
            pl.BlockSpec((H, 128), lambda i: (0, 0)),
            pl.BlockSpec((1, H), lambda i: (0, 0)),
            pl.BlockSpec((H, H), lambda i: (0, 0)),
            pl.BlockSpec((1, H), lambda i: (0, 0)),
        ],
        out_specs=[
            pl.BlockSpec((BLK, H), lambda i: (i, 0)),
            pl.BlockSpec((BLK, H), lambda i: (i, 0)),
            pl.BlockSpec((2, H), lambda i: (0, 0)),
        ],
        out_shape=[
            jax.ShapeDtypeStruct((N, H), jnp.float32),
            jax.ShapeDtypeStruct((N, H), jnp.float32),
            jax.ShapeDtypeStruct((2, H), jnp.float32),
        ],
    )(xT, W_nm, row2(b_nm), W_v2v, row2(b_v2v))

    sc_gather = functools.partial(
        pl.kernel,
        mesh=plsc.VectorSubcoreMesh(core_axis_name="c", subcore_axis_name="s",
                                    num_cores=SC_NC, num_subcores=SC_NS),
        out_type=jax.ShapeDtypeStruct((E_EDGES, 2 * H), jnp.float32),
        scratch_types=[
            pltpu.VMEM((SC_CHUNK,), jnp.int32),
            pltpu.VMEM((SC_CHUNK,), jnp.int32),
            pltpu.VMEM((SC_CHUNK, 2 * H), jnp.float32),
            pltpu.VMEM((SC_CHUNK, 2 * H), jnp.float32),
            pltpu.SemaphoreType.DMA,
            pltpu.SemaphoreType.DMA,
        ],
    )(_make_sc_gather_body(ROWS_PER_W, ROWS_PER_W // SC_CHUNK))
    g_rows = sc_gather(g_tabT, idx_flat)
    g3 = g_rows.reshape(N, K, 2 * H)

    BLK4 = 400
    GRID4 = N // BLK4
    zT, zstats = pl.pallas_call(
        _k4_combine,
        grid=(GRID4,),
        in_specs=[
            pl.BlockSpec((BLK4, K, 2 * H), lambda i: (i, 0, 0)),
            pl.BlockSpec((NE, 1, BLK4, K), lambda i: (0, i, 0, 0)),
            pl.BlockSpec((BLK4, H), lambda i: (i, 0)),
            pl.BlockSpec((BLK4, H), lambda i: (i, 0)),
            pl.BlockSpec((2, H), lambda i: (0, 0)),
            pl.BlockSpec((2 * H, NE), lambda i: (0, 0)),
            pl.BlockSpec((H, 2 * H), lambda i: (0, 0)),
            pl.BlockSpec((1, H), lambda i: (0, 0)),
            pl.BlockSpec((1, H), lambda i: (0, 0)),
            pl.BlockSpec((1, H), lambda i: (0, 0)),
            pl.BlockSpec((128, H), lambda i: (0, 0)),
            pl.BlockSpec((1, 128), lambda i: (0, 0)),
        ],
        out_specs=[
            pl.BlockSpec((BLK4, 128), lambda i: (i, 0)),
            pl.BlockSpec((2, 128), lambda i: (0, 0)),
        ],
        out_shape=[
            jax.ShapeDtypeStruct((N, 128), jnp.float32),
            jax.ShapeDtypeStruct((2, 128), jnp.float32),
        ],
    )(g3, ef.reshape(NE, GRID4, BLK4, K), nnodeT, y2T, stats,
      jnp.pad(Wm_f2v[:, H:], ((0, H), (0, 0))),
      jnp.pad(W2_f2v, ((0, 0), (0, H))), row2(b2_f2v),
      row2(g_v2v), row2(bt_v2v), Wc1, row2(bc1))

    zzero = jnp.zeros((2, 128), jnp.float32)
    out = pl.pallas_call(
        _k5_final,
        grid=(GRID,),
        in_specs=[
            pl.BlockSpec((BLK, 128), lambda i: (i, 0)),
            pl.BlockSpec((2, 128), lambda i: (0, 0)),
            pl.BlockSpec((2, 128), lambda i: (0, 0)),
            pl.BlockSpec((1, 128), lambda i: (0, 0)),
            pl.BlockSpec((1, 128), lambda i: (0, 0)),
            pl.BlockSpec((1, 128), lambda i: (0, 0)),
            pl.BlockSpec((1, 1), lambda i: (0, 0)),
        ],
        out_specs=pl.BlockSpec((BLK, 1), lambda i: (i, 0)),
        out_shape=jax.ShapeDtypeStruct((N, 1), jnp.float32),
    )(zT, zstats, zzero, row2(gc), row2(btc), Wc2, bc2.reshape(1, 1))

    return out.reshape(1, 1, N, 1)


# speedup vs baseline: 11.9988x; 1.0007x over previous
import functools

import jax
import jax.numpy as jnp
from jax import lax
from jax.experimental import pallas as pl
from jax.experimental.pallas import tpu as pltpu
from jax.experimental.pallas import tpu_sc as plsc

N = 50000
F = 25000
K = 16
NE = 4
H = 64
EPS = 1e-5

BLK = 1000
GRID = N // BLK

SC_NC = 2
SC_NS = 16
SC_NW = SC_NC * SC_NS
E_EDGES = N * K
ROWS_PER_W = E_EDGES // SC_NW
SC_CHUNK = 200


def _k1_factor(hop_ref, wfm_ref, bfm_ref, gfm_ref, btfm_ref,
               w1_ref, b1_ref, wmh_ref, bm_ref, out_ref):
    y = jnp.dot(wfm_ref[...], hop_ref[...],
                preferred_element_type=jnp.float32) + bfm_ref[...]
    m = jnp.mean(y, axis=1, keepdims=True)
    v = jnp.mean((y - m) ** 2, axis=1, keepdims=True)
    nhop = jax.nn.relu((y - m) * lax.rsqrt(v + EPS) * gfm_ref[...]
                       + btfm_ref[...])
    h = jax.nn.relu(jnp.dot(w1_ref[...], nhop,
                            preferred_element_type=jnp.float32) + b1_ref[...])
    out_ref[...] = lax.dot_general(h, wmh_ref[...], (((0,), (1,)), ((), ())),
                                   preferred_element_type=jnp.float32) + bm_ref[...]


def _k2_node(xt_ref, wnm_ref, bnm_ref, wv2v_ref, bv2v_ref,
             nnode_ref, y2_ref, stats_ref):
    nn = jax.nn.relu(
        lax.dot_general(xt_ref[...], wnm_ref[...], (((1,), (1,)), ((), ())),
                        preferred_element_type=jnp.float32) + bnm_ref[...])
    y2 = lax.dot_general(nn, wv2v_ref[...], (((1,), (1,)), ((), ())),
                         preferred_element_type=jnp.float32) + bv2v_ref[...]
    nnode_ref[...] = nn
    y2_ref[...] = y2
    st = jnp.concatenate([jnp.sum(y2, axis=0, keepdims=True),
                          jnp.sum(y2 * y2, axis=0, keepdims=True)], axis=0)

    @pl.when(pl.program_id(0) == 0)
    def _():
        stats_ref[...] = st

    @pl.when(pl.program_id(0) != 0)
    def _():
        stats_ref[...] += st


def _make_sc_gather_body(rows_per_w, iters):
    def _sc_gather_body(table_ref, idx_ref, out_ref,
                        idx0, idx1, rows0, rows1, sem0, sem1):
        wid = lax.axis_index("s") * SC_NC + lax.axis_index("c")
        base = wid * rows_per_w

        def _step(j, cur_idx, cur_rows, cur_sem, nxt_idx, nxt_rows, nxt_sem):
            @pl.when(j + 1 < iters)
            def _():
                off_n = base + (j + 1) * SC_CHUNK
                pltpu.sync_copy(idx_ref.at[pl.ds(off_n, SC_CHUNK)], nxt_idx)
                pltpu.async_copy(table_ref.at[nxt_idx], nxt_rows, nxt_sem)
            pltpu.make_async_copy(table_ref.at[cur_idx], cur_rows,
                                  cur_sem).wait()
            pltpu.sync_copy(cur_rows, out_ref.at[pl.ds(base + j * SC_CHUNK,
                                                       SC_CHUNK)])

        def body(j, carry):
            @pl.when(j % 2 == 0)
            def _():
                _step(j, idx0, rows0, sem0, idx1, rows1, sem1)

            @pl.when(j % 2 == 1)
            def _():
                _step(j, idx1, rows1, sem1, idx0, rows0, sem0)

            return carry

        pltpu.sync_copy(idx_ref.at[pl.ds(base, SC_CHUNK)], idx0)
        pltpu.async_copy(table_ref.at[idx0], rows0, sem0)
        lax.fori_loop(0, iters, body, 0)

    return _sc_gather_body


def _k4_combine(g3_ref, ef_ref, nnode_ref, y2_ref, stats_ref,
                wme_ref, w2_ref, b2_ref, gv_ref, btv_ref,
                wc1_ref, bc1_ref, z_ref, zstats_ref):
    e2 = lax.dot_general(ef_ref[:, 0], wme_ref[...], (((0,), (1,)), ((), ())),
                         preferred_element_type=jnp.float32)
    msg = jax.nn.relu(g3_ref[...] + e2)
    acc = jnp.max(msg, axis=1)
    nv = lax.dot_general(acc, w2_ref[...], (((1,), (1,)), ((), ())),
                         preferred_element_type=jnp.float32) + b2_ref[...]
    st = stats_ref[...]
    m = st[0:1, :] / N
    v = st[1:2, :] / N - m * m
    inv = lax.rsqrt(v + EPS)
    y2 = y2_ref[...]
    nf = jax.nn.relu((y2 - m) * inv * gv_ref[...] + btv_ref[...]) + nv
    nnode2 = nnode_ref[...] + nf
    z = lax.dot_general(nnode2, wc1_ref[...], (((1,), (1,)), ((), ())),
                        preferred_element_type=jnp.float32) + bc1_ref[...]
    z_ref[...] = z
    st2 = jnp.concatenate([jnp.sum(z, axis=0, keepdims=True),
                           jnp.sum(z * z, axis=0, keepdims=True)], axis=0)

    @pl.when(pl.program_id(0) == 0)
    def _():
        zstats_ref[...] = st2

    @pl.when(pl.program_id(0) != 0)
    def _():
        zstats_ref[...] += st2


def _k5_final(z_ref, zsta_ref, zstb_ref, gc_ref, btc_ref, wc2_ref, bc2_ref,
              out_ref):
    st = zsta_ref[...] + zstb_ref[...]
    m = st[0:1, :] / N
    v = st[1:2, :] / N - m * m
    inv = lax.rsqrt(v + EPS)
    hcls = jax.nn.relu((z_ref[...] - m) * inv * gc_ref[...] + btc_ref[...])
    o = jnp.sum(hcls * wc2_ref[...], axis=1, keepdims=True)
    out_ref[...] = o + bc2_ref[...]


def kernel(node_feature, hop_features, etype_f2v, etype_v2f, W_nm, b_nm, W_fm, b_fm, g_fm, bt_fm, W_v2v, b_v2v, g_v2v, bt_v2v, W_f2f, b_f2f, g_f2f, bt_f2f, W1_f2v, b1_f2v, Wm_f2v, bm_f2v, W2_f2v, b2_f2v, W1_v2f, b1_v2f, Wm_v2f, bm_v2f, W2_v2f, b2_v2f, Wc1, bc1, gc, btc, Wc2, bc2, nn_idx_f2v, nn_idx_v2f):
    xT = node_feature[0, :, :, 0].T
    hop = hop_features[0, 0, :, :, 0]
    idx = nn_idx_f2v[0, 0].astype(jnp.int32)
    idx_flat = idx.reshape(E_EDGES)
    ef = etype_f2v[0, 0]

    row2 = lambda a: a.reshape(1, -1)

    g_tabT = pl.pallas_call(
        _k1_factor,
        out_shape=jax.ShapeDtypeStruct((F, 2 * H), jnp.float32),
    )(hop, W_fm, b_fm.reshape(H, 1), g_fm.reshape(H, 1), bt_fm.reshape(H, 1),
      W1_f2v, b1_f2v.reshape(H, 1),
      jnp.pad(Wm_f2v[:, :H], ((0, H), (0, 0))),
      row2(jnp.pad(bm_f2v, (0, H))))

    nnodeT, y2T, stats = pl.pallas_call(
        _k2_node,
        grid=(GRID,),
        in_specs=[
            pl.BlockSpec((BLK, 128), lambda i: (i, 0)),
            pl.BlockSpec((H, 128), lambda i: (0, 0)),
            pl.BlockSpec((1, H), lambda i: (0, 0)),
            pl.BlockSpec((H, H), lambda i: (0, 0)),
            pl.BlockSpec((1, H), lambda i: (0, 0)),
        ],
        out_specs=[
            pl.BlockSpec((BLK, H), lambda i: (i, 0)),
            pl.BlockSpec((BLK, H), lambda i: (i, 0)),
            pl.BlockSpec((2, H), lambda i: (0, 0)),
        ],
        out_shape=[
            jax.ShapeDtypeStruct((N, H), jnp.float32),
            jax.ShapeDtypeStruct((N, H), jnp.float32),
            jax.ShapeDtypeStruct((2, H), jnp.float32),
        ],
    )(xT, W_nm, row2(b_nm), W_v2v, row2(b_v2v))

    sc_gather = functools.partial(
        pl.kernel,
        mesh=plsc.VectorSubcoreMesh(core_axis_name="c", subcore_axis_name="s",
                                    num_cores=SC_NC, num_subcores=SC_NS),
        out_type=jax.ShapeDtypeStruct((E_EDGES, 2 * H), jnp.float32),
        scratch_types=[
            pltpu.VMEM((SC_CHUNK,), jnp.int32),
            pltpu.VMEM((SC_CHUNK,), jnp.int32),
            pltpu.VMEM((SC_CHUNK, 2 * H), jnp.float32),
            pltpu.VMEM((SC_CHUNK, 2 * H), jnp.float32),
            pltpu.SemaphoreType.DMA,
            pltpu.SemaphoreType.DMA,
        ],
    )(_make_sc_gather_body(ROWS_PER_W, ROWS_PER_W // SC_CHUNK))
    g_rows = sc_gather(g_tabT, idx_flat)
    g3 = g_rows.reshape(N, K, 2 * H)

    BLK4 = 400
    GRID4 = N // BLK4
    zT, zstats = pl.pallas_call(
        _k4_combine,
        grid=(GRID4,),
        in_specs=[
            pl.BlockSpec((BLK4, K, 2 * H), lambda i: (i, 0, 0)),
            pl.BlockSpec((NE, 1, BLK4, K), lambda i: (0, i, 0, 0)),
            pl.BlockSpec((BLK4, H), lambda i: (i, 0)),
            pl.BlockSpec((BLK4, H), lambda i: (i, 0)),
            pl.BlockSpec((2, H), lambda i: (0, 0)),
            pl.BlockSpec((2 * H, NE), lambda i: (0, 0)),
            pl.BlockSpec((H, 2 * H), lambda i: (0, 0)),
            pl.BlockSpec((1, H), lambda i: (0, 0)),
            pl.BlockSpec((1, H), lambda i: (0, 0)),
            pl.BlockSpec((1, H), lambda i: (0, 0)),
            pl.BlockSpec((128, H), lambda i: (0, 0)),
            pl.BlockSpec((1, 128), lambda i: (0, 0)),
        ],
        out_specs=[
            pl.BlockSpec((BLK4, 128), lambda i: (i, 0)),
            pl.BlockSpec((2, 128), lambda i: (0, 0)),
        ],
        out_shape=[
            jax.ShapeDtypeStruct((N, 128), jnp.float32),
            jax.ShapeDtypeStruct((2, 128), jnp.float32),
        ],
    )(g3, ef.reshape(NE, GRID4, BLK4, K), nnodeT, y2T, stats,
      jnp.pad(Wm_f2v[:, H:], ((0, H), (0, 0))),
      jnp.pad(W2_f2v, ((0, 0), (0, H))), row2(b2_f2v),
      row2(g_v2v), row2(bt_v2v), Wc1, row2(bc1))

    zzero = jnp.zeros((2, 128), jnp.float32)
    out = pl.pallas_call(
        _k5_final,
        grid=(GRID,),
        in_specs=[
            pl.BlockSpec((BLK, 128), lambda i: (i, 0)),
            pl.BlockSpec((2, 128), lambda i: (0, 0)),
            pl.BlockSpec((2, 128), lambda i: (0, 0)),
            pl.BlockSpec((1, 128), lambda i: (0, 0)),
            pl.BlockSpec((1, 128), lambda i: (0, 0)),
            pl.BlockSpec((1, 128), lambda i: (0, 0)),
            pl.BlockSpec((1, 1), lambda i: (0, 0)),
        ],
        out_specs=pl.BlockSpec((BLK, 1), lambda i: (i, 0)),
        out_shape=jax.ShapeDtypeStruct((N, 1), jnp.float32),
    )(zT, zstats, zzero, row2(gc), row2(btc), Wc2, bc2.reshape(1, 1))

    return out.reshape(1, 1, N, 1)
